# baseline (speedup 1.0000x reference)
import functools

import jax
import jax.numpy as jnp
from jax import lax
from jax.experimental import pallas as pl
from jax.experimental.pallas import tpu as pltpu
from jax.experimental.pallas import tpu_sc as plsc

_DIM = 64
_GAMMA = 12.0
_C = 128
_L = 16
_U = 8


@functools.lru_cache(maxsize=4)
def _build_sc_call(B, V_ent, V_rel):
    info = plsc.get_sparse_core_info()
    nw = info.num_cores * info.num_subcores
    rows_w = B // nw
    n_chunks = rows_w // _C
    assert rows_w % _C == 0 and B % nw == 0

    mesh = plsc.VectorSubcoreMesh(core_axis_name="c", subcore_axis_name="s")

    @functools.partial(
        pl.kernel,
        mesh=mesh,
        out_type=jax.ShapeDtypeStruct((B,), jnp.float32),
        compiler_params=pltpu.CompilerParams(
            needs_layout_passes=False, use_tc_tiling_on_sc=False),
        scratch_types=(
            [pltpu.VMEM((n_chunks, _C), jnp.int32) for _ in range(3)]
            + [pltpu.VMEM((2, _C, _DIM), jnp.float32) for _ in range(5)]
            + [pltpu.VMEM((_C,), jnp.float32),
               pltpu.SemaphoreType.DMA,
               pltpu.SemaphoreType.DMA]
        ),
    )
    def sc_call(ih_hbm, ir_hbm, it_hbm, ent_hbm, rel_hbm, entp_hbm, out_hbm,
                ih_v, ir_v, it_v, h_v, r_v, t_v, hp_v, tp_v, out_v,
                sem0, sem1):
        wid = lax.axis_index("s") * info.num_cores + lax.axis_index("c")
        base_w = wid * rows_w
        sems = (sem0, sem1)

        for k in range(n_chunks):
            b = base_w + k * _C
            pltpu.sync_copy(ih_hbm.at[pl.ds(b, _C)], ih_v.at[k])
            pltpu.sync_copy(ir_hbm.at[pl.ds(b, _C)], ir_v.at[k])
            pltpu.sync_copy(it_hbm.at[pl.ds(b, _C)], it_v.at[k])

        def fire(k, slot):
            sem = sems[slot]
            return [
                pltpu.async_copy(ent_hbm.at[ih_v.at[k]], h_v.at[slot], sem),
                pltpu.async_copy(rel_hbm.at[ir_v.at[k]], r_v.at[slot], sem),
                pltpu.async_copy(ent_hbm.at[it_v.at[k]], t_v.at[slot], sem),
                pltpu.async_copy(entp_hbm.at[ih_v.at[k]], hp_v.at[slot], sem),
                pltpu.async_copy(entp_hbm.at[it_v.at[k]], tp_v.at[slot], sem),
            ]

        def compute(slot, k):
            hs, rs, ts = h_v.at[slot], r_v.at[slot], t_v.at[slot]
            hps, tps = hp_v.at[slot], tp_v.at[slot]

            def group(g, carry):
                rows = g * _L + lax.iota(jnp.int32, _L)
                zf = jnp.zeros((_L,), jnp.float32)
                zi = jnp.zeros((_L,), jnp.int32)

                def p1(j, c):
                    tph, tpt, rn, cols = c
                    for u in range(_U):
                        col = cols + u
                        tpv = plsc.load_gather(tps, [rows, col])
                        hv = plsc.load_gather(hs, [rows, col])
                        tv = plsc.load_gather(ts, [rows, col])
                        rv = plsc.load_gather(rs, [rows, col])
                        tph = tph + tpv * hv
                        tpt = tpt + tpv * tv
                        rn = rn + rv * rv
                    return tph, tpt, rn, cols + _U
                tph, tpt, rn, _ = lax.fori_loop(
                    0, _DIM // _U, p1, (zf, zf, zf, zi))

                dtp = tph - tpt
                x = jnp.maximum(rn, 1.0)
                yi = jnp.int32(0x5F3759DF) - (plsc.bitcast(x, jnp.int32) >> 1)
                y = plsc.bitcast(yi, jnp.float32)
                for _ in range(3):
                    y = y * (1.5 - 0.5 * x * y * y)
                nrm = x * y
                rscale = jnp.where(rn > 1.0, 1.0 / (nrm + 1e-7), 1.0)

                def p2(j, c):
                    acc, cols = c
                    for u in range(_U):
                        col = cols + u
                        hpv = plsc.load_gather(hps, [rows, col])
                        hv = plsc.load_gather(hs, [rows, col])
                        tv = plsc.load_gather(ts, [rows, col])
                        rv = plsc.load_gather(rs, [rows, col])
                        s = hpv * dtp + (hv - tv) + rv * rscale
                        acc = acc + jnp.abs(s)
                    return acc, cols + _U
                acc, _ = lax.fori_loop(0, _DIM // _U, p2, (zf, zi))

                out_v[pl.ds(g * _L, _L)] = acc - _GAMMA
                return carry
            lax.fori_loop(0, _C // _L, group, 0)
            pltpu.sync_copy(out_v, out_hbm.at[pl.ds(base_w + k * _C, _C)])

        pend = fire(0, 0)
        for k in range(n_chunks):
            nxt = fire(k + 1, (k + 1) % 2) if k + 1 < n_chunks else []
            for cp in pend:
                cp.wait()
            compute(k % 2, k)
            pend = nxt

    return sc_call


def kernel(pos_sample, ent_embd, rel_embd, ent_p):
    B = pos_sample.shape[0]
    idx = pos_sample.astype(jnp.int32)
    sc_call = _build_sc_call(B, ent_embd.shape[0], rel_embd.shape[0])
    score = sc_call(idx[:, 0], idx[:, 1], idx[:, 2],
                    ent_embd, rel_embd, ent_p)
    return score[:, None]
